# baseline (speedup 1.0000x reference)
import jax
import jax.numpy as jnp
from jax import lax
from jax.experimental import pallas as pl
from jax.experimental.pallas import tpu as pltpu
from jax.experimental.pallas import tpu_sc as plsc

N = 320000
D = 128
L = 10000

NC = 2
NS = 16
LANES = 16
NW = NC * NS
JOBS_PER_W = 2
JOBS = NW * JOBS_PER_W
W = (-(-L // JOBS) + 7) // 8 * 8
L_PAD = JOBS * W
CB = D // LANES
ACC_W = W * 3 * D
OUT_WORDS = L * 3 * D
FULL_JOBS = L // W
REM_WORDS = (L - FULL_JOBS * W) * 3 * D

CHUNK = 256
GPC = CHUNK // LANES
NB = 2

_INF = float("inf")


def _sc_body(x_hbm, lab_hbm, cnt_hbm, jinfo_hbm, out_hbm,
             rows0, rows1, labs0, labs1, acc_v, racc_v, cnt_v, jinfo_v,
             run_s, sem0, sem1):
    wid = lax.axis_index("s") * NC + lax.axis_index("c")
    zeros = jnp.zeros((LANES,), jnp.float32)
    rows_refs = (rows0, rows1)
    labs_refs = (labs0, labs1)
    sems = (sem0, sem1)

    def ld_racc():
        s_acc = [racc_v[pl.ds(cb * LANES, LANES)] for cb in range(CB)]
        n_acc = [racc_v[pl.ds(D + cb * LANES, LANES)] for cb in range(CB)]
        x_acc = [racc_v[pl.ds(2 * D + cb * LANES, LANES)] for cb in range(CB)]
        return s_acc, n_acc, x_acc

    def st_racc(s_acc, n_acc, x_acc):
        for cb in range(CB):
            racc_v[pl.ds(cb * LANES, LANES)] = s_acc[cb]
            racc_v[pl.ds(D + cb * LANES, LANES)] = n_acc[cb]
            racc_v[pl.ds(2 * D + cb * LANES, LANES)] = x_acc[cb]

    def job_body(jj, _):
        job = wid * JOBS_PER_W + jj
        l_lo = job * W

        pltpu.sync_copy(jinfo_hbm.at[job], jinfo_v)
        jv = jinfo_v[...]
        r0 = jv[0]
        r1 = jv[1]
        base = (r0 // 16) * 16
        end = ((r1 + 15) // 16) * 16
        nch = (end - base + CHUNK - 1) // CHUNK

        run_s[0] = jnp.int32(-1)

        def zero_body(i, c):
            for u in range(16):
                acc_v[pl.ds(i * 256 + u * LANES, LANES)] = zeros
            return c

        lax.fori_loop(0, ACC_W // 256, zero_body, 0)

        def flush(run_lab, s_acc, n_acc, x_acc):
            off = (run_lab - l_lo) * (3 * D)
            for cb in range(CB):
                acc_v[pl.ds(off + cb * LANES, LANES)] = s_acc[cb]
                acc_v[pl.ds(off + D + cb * LANES, LANES)] = n_acc[cb]
                acc_v[pl.ds(off + 2 * D + cb * LANES, LANES)] = x_acc[cb]

        def process_group(rows_ref, labs_ref, g):
            run0 = run_s[0]
            lv = labs_ref[pl.ds(g * 16, 16)]
            all_same = jnp.logical_and(lv[0] == run0, lv[15] == run0)

            @pl.when(all_same)
            def _():
                s_acc, n_acc, x_acc = ld_racc()
                for i in range(16):
                    v = [rows_ref[g * 16 + i, pl.ds(cb * LANES, LANES)]
                         for cb in range(CB)]
                    s_acc = [s_acc[cb] + v[cb] for cb in range(CB)]
                    n_acc = [jnp.minimum(n_acc[cb], v[cb])
                             for cb in range(CB)]
                    x_acc = [jnp.maximum(x_acc[cb], v[cb])
                             for cb in range(CB)]
                st_racc(s_acc, n_acc, x_acc)

            @pl.when(jnp.logical_not(all_same))
            def _():
                lab2 = lv[15]
                li = [lv[i] for i in range(16)]
                eq2 = [jnp.logical_or(li[i] == run0, li[i] == lab2)
                       for i in range(16)]
                while len(eq2) > 1:
                    eq2 = [jnp.logical_and(eq2[j], eq2[j + 1])
                           for j in range(0, len(eq2) - 1, 2)] + (
                               [eq2[-1]] if len(eq2) % 2 else [])
                one_b = eq2[0]
                kv = [jnp.where(li[i] == lab2, jnp.int32(i), jnp.int32(16))
                      for i in range(16)]
                while len(kv) > 1:
                    kv = [jnp.minimum(kv[j], kv[j + 1])
                          for j in range(0, len(kv) - 1, 2)] + (
                              [kv[-1]] if len(kv) % 2 else [])
                k = kv[0]

                @pl.when(one_b)
                def _():
                    s_acc, n_acc, x_acc = ld_racc()

                    def acc_row(i, c):
                        sa, na, xa = c
                        v = [rows_ref[g * 16 + i, pl.ds(cb * LANES, LANES)]
                             for cb in range(CB)]
                        sa = tuple(sa[cb] + v[cb] for cb in range(CB))
                        na = tuple(jnp.minimum(na[cb], v[cb])
                                   for cb in range(CB))
                        xa = tuple(jnp.maximum(xa[cb], v[cb])
                                   for cb in range(CB))
                        return sa, na, xa

                    s_acc, n_acc, x_acc = lax.fori_loop(
                        0, k, acc_row,
                        (tuple(s_acc), tuple(n_acc), tuple(x_acc)))

                    @pl.when(jnp.logical_and(run0 >= l_lo, run0 < l_lo + W))
                    def _(s_acc=s_acc, n_acc=n_acc, x_acc=x_acc):
                        flush(run0, s_acc, n_acc, x_acc)

                    vk = tuple(rows_ref[g * 16 + k, pl.ds(cb * LANES, LANES)]
                               for cb in range(CB))
                    s_acc, n_acc, x_acc = lax.fori_loop(
                        k + 1, 16, acc_row, (vk, vk, vk))
                    st_racc(s_acc, n_acc, x_acc)
                    run_s[0] = lab2

                @pl.when(jnp.logical_not(one_b))
                def _():
                    _general_group(rows_ref, li, g)

        def _general_group(rows_ref, li, g):
                run_lab = run_s[0]
                s_acc, n_acc, x_acc = ld_racc()
                for i in range(16):
                    lab = li[i]
                    is_new = lab != run_lab
                    do_flush = jnp.logical_and(
                        is_new,
                        jnp.logical_and(run_lab >= l_lo, run_lab < l_lo + W))

                    @pl.when(do_flush)
                    def _(run_lab=run_lab, s_acc=s_acc, n_acc=n_acc,
                          x_acc=x_acc):
                        flush(run_lab, s_acc, n_acc, x_acc)

                    v = [rows_ref[g * 16 + i, pl.ds(cb * LANES, LANES)]
                         for cb in range(CB)]
                    s_acc = [jnp.where(is_new, v[cb], s_acc[cb] + v[cb])
                             for cb in range(CB)]
                    n_acc = [
                        jnp.minimum(jnp.where(is_new, _INF, n_acc[cb]), v[cb])
                        for cb in range(CB)]
                    x_acc = [
                        jnp.maximum(jnp.where(is_new, -_INF, x_acc[cb]), v[cb])
                        for cb in range(CB)]
                    run_lab = jnp.where(is_new, lab, run_lab)
                st_racc(s_acc, n_acc, x_acc)
                run_s[0] = run_lab

        def chunk_start(i):
            return jnp.minimum(base + i * CHUNK, N - CHUNK)

        def fire(i, b):
            s = chunk_start(i)
            pltpu.async_copy(x_hbm.at[pl.ds(s, CHUNK)], rows_refs[b], sems[b])
            pltpu.async_copy(lab_hbm.at[pl.ds(s, CHUNK)], labs_refs[b],
                             sems[b])

        def wait_slot(b):
            pltpu.make_async_copy(
                x_hbm.at[pl.ds(0, CHUNK)], rows_refs[b], sems[b]).wait()
            pltpu.make_async_copy(
                lab_hbm.at[pl.ds(0, CHUNK)], labs_refs[b], sems[b]).wait()

        for b in range(NB):
            @pl.when(b < nch)
            def _(b=b):
                fire(b, b)

        def super_body(ss, c):
            for b in range(NB):
                i = ss * NB + b

                @pl.when(i < nch)
                def _(i=i, b=b):
                    wait_slot(b)
                    g0 = (base + i * CHUNK - chunk_start(i)) // 16

                    def gbody(g, cc, b=b):
                        process_group(rows_refs[b], labs_refs[b], g)
                        return cc

                    lax.fori_loop(g0, GPC, gbody, 0)

                    @pl.when(i + NB < nch)
                    def _(i=i, b=b):
                        fire(i + NB, b)
            return c

        lax.fori_loop(0, (nch + NB - 1) // NB, super_body, 0)

        run_lab = run_s[0]

        @pl.when(jnp.logical_and(run_lab >= l_lo, run_lab < l_lo + W))
        def _():
            s_acc, n_acc, x_acc = ld_racc()
            flush(run_lab, s_acc, n_acc, x_acc)

        pltpu.sync_copy(cnt_hbm.at[pl.ds(l_lo, W)], cnt_v)

        def div_body(g, c):
            cnt = cnt_v[pl.ds(g * 16, 16)]
            rcp = 1.0 / jnp.maximum(cnt.astype(jnp.float32), 1.0)
            for i in range(16):
                r = rcp[i]
                off = (g * 16 + i) * (3 * D)
                for cb in range(CB):
                    sl = pl.ds(off + cb * LANES, LANES)
                    acc_v[sl] = acc_v[sl] * r
            return c

        lax.fori_loop(0, W // 16, div_body, 0)

        @pl.when(job < FULL_JOBS)
        def _():
            pltpu.sync_copy(acc_v, out_hbm.at[pl.ds(job * ACC_W, ACC_W)])

        @pl.when(job == FULL_JOBS)
        def _():
            pltpu.sync_copy(acc_v.at[pl.ds(0, REM_WORDS)],
                            out_hbm.at[pl.ds(FULL_JOBS * ACC_W, REM_WORDS)])
        return 0

    lax.fori_loop(0, JOBS_PER_W, job_body, 0)


def kernel(input, labels, labelcount):
    counts_pad = jnp.concatenate(
        [labelcount, jnp.ones((L_PAD - L,), jnp.int32)])
    starts = jnp.concatenate(
        [jnp.zeros((1,), jnp.int32), jnp.cumsum(labelcount, dtype=jnp.int32)])
    bnd = jnp.minimum(jnp.arange(0, L_PAD + W, W, dtype=jnp.int32), L)
    js = starts[bnd]
    jinfo = (jnp.zeros((JOBS, 16), jnp.int32)
             .at[:, 0].set(js[:-1])
             .at[:, 1].set(js[1:]))

    mesh = plsc.VectorSubcoreMesh(core_axis_name="c", subcore_axis_name="s")
    out_flat = pl.kernel(
        _sc_body,
        out_type=jax.ShapeDtypeStruct((OUT_WORDS,), jnp.float32),
        mesh=mesh,
        scratch_types=[
            pltpu.VMEM((CHUNK, D), jnp.float32),
            pltpu.VMEM((CHUNK, D), jnp.float32),
            pltpu.VMEM((CHUNK,), jnp.int32),
            pltpu.VMEM((CHUNK,), jnp.int32),
            pltpu.VMEM((ACC_W,), jnp.float32),
            pltpu.VMEM((3 * D,), jnp.float32),
            pltpu.VMEM((W,), jnp.int32),
            pltpu.VMEM((16,), jnp.int32),
            pltpu.SMEM((8,), jnp.int32),
            pltpu.SemaphoreType.DMA,
            pltpu.SemaphoreType.DMA,
        ],
    )(input, labels, counts_pad, jinfo)
    return out_flat.reshape(L, 3, D)


# speedup vs baseline: 1.0073x; 1.0073x over previous
import jax
import jax.numpy as jnp
from jax import lax
from jax.experimental import pallas as pl
from jax.experimental.pallas import tpu as pltpu
from jax.experimental.pallas import tpu_sc as plsc

N = 320000
D = 128
L = 10000

NC = 2
NS = 16
LANES = 16
NW = NC * NS
JOBS_PER_W = 2
JOBS = NW * JOBS_PER_W
W = (-(-L // JOBS) + 7) // 8 * 8
L_PAD = JOBS * W
CB = D // LANES
ACC_W = W * 3 * D
OUT_WORDS = L * 3 * D
FULL_JOBS = L // W
REM_WORDS = (L - FULL_JOBS * W) * 3 * D

CHUNK = 128
GPC = CHUNK // LANES
NB = 2

_INF = float("inf")


def _sc_body(x_hbm, lab_hbm, cnt_hbm, jinfo_hbm, out_hbm,
             rows0, rows1, labs0, labs1, acc_v, racc_v, cnt_v, jinfo_v,
             run_s, sem0, sem1):
    wid = lax.axis_index("s") * NC + lax.axis_index("c")
    zeros = jnp.zeros((LANES,), jnp.float32)
    rows_refs = (rows0, rows1)
    labs_refs = (labs0, labs1)
    sems = (sem0, sem1)

    def ld_racc():
        s_acc = [racc_v[pl.ds(cb * LANES, LANES)] for cb in range(CB)]
        n_acc = [racc_v[pl.ds(D + cb * LANES, LANES)] for cb in range(CB)]
        x_acc = [racc_v[pl.ds(2 * D + cb * LANES, LANES)] for cb in range(CB)]
        return s_acc, n_acc, x_acc

    def st_racc(s_acc, n_acc, x_acc):
        for cb in range(CB):
            racc_v[pl.ds(cb * LANES, LANES)] = s_acc[cb]
            racc_v[pl.ds(D + cb * LANES, LANES)] = n_acc[cb]
            racc_v[pl.ds(2 * D + cb * LANES, LANES)] = x_acc[cb]

    def job_body(jj, _):
        job = wid * JOBS_PER_W + jj
        l_lo = job * W

        pltpu.sync_copy(jinfo_hbm.at[job], jinfo_v)
        jv = jinfo_v[...]
        r0 = jv[0]
        r1 = jv[1]
        base = (r0 // 16) * 16
        end = ((r1 + 15) // 16) * 16
        nch = (end - base + CHUNK - 1) // CHUNK

        run_s[0] = jnp.int32(-1)

        def zero_body(i, c):
            for u in range(16):
                acc_v[pl.ds(i * 256 + u * LANES, LANES)] = zeros
            return c

        lax.fori_loop(0, ACC_W // 256, zero_body, 0)

        def flush(run_lab, s_acc, n_acc, x_acc):
            off = (run_lab - l_lo) * (3 * D)
            for cb in range(CB):
                acc_v[pl.ds(off + cb * LANES, LANES)] = s_acc[cb]
                acc_v[pl.ds(off + D + cb * LANES, LANES)] = n_acc[cb]
                acc_v[pl.ds(off + 2 * D + cb * LANES, LANES)] = x_acc[cb]

        def process_group(rows_ref, labs_ref, g):
            run0 = run_s[0]
            lv = labs_ref[pl.ds(g * 16, 16)]
            all_same = jnp.logical_and(lv[0] == run0, lv[15] == run0)

            @pl.when(all_same)
            def _():
                s_acc, n_acc, x_acc = ld_racc()
                for i in range(16):
                    v = [rows_ref[g * 16 + i, pl.ds(cb * LANES, LANES)]
                         for cb in range(CB)]
                    s_acc = [s_acc[cb] + v[cb] for cb in range(CB)]
                    n_acc = [jnp.minimum(n_acc[cb], v[cb])
                             for cb in range(CB)]
                    x_acc = [jnp.maximum(x_acc[cb], v[cb])
                             for cb in range(CB)]
                st_racc(s_acc, n_acc, x_acc)

            @pl.when(jnp.logical_not(all_same))
            def _():
                lab2 = lv[15]
                li = [lv[i] for i in range(16)]
                eq2 = [jnp.logical_or(li[i] == run0, li[i] == lab2)
                       for i in range(16)]
                while len(eq2) > 1:
                    eq2 = [jnp.logical_and(eq2[j], eq2[j + 1])
                           for j in range(0, len(eq2) - 1, 2)] + (
                               [eq2[-1]] if len(eq2) % 2 else [])
                one_b = eq2[0]
                kv = [jnp.where(li[i] == lab2, jnp.int32(i), jnp.int32(16))
                      for i in range(16)]
                while len(kv) > 1:
                    kv = [jnp.minimum(kv[j], kv[j + 1])
                          for j in range(0, len(kv) - 1, 2)] + (
                              [kv[-1]] if len(kv) % 2 else [])
                k = kv[0]

                @pl.when(one_b)
                def _():
                    s_acc, n_acc, x_acc = ld_racc()

                    def acc_row(i, c):
                        sa, na, xa = c
                        v = [rows_ref[g * 16 + i, pl.ds(cb * LANES, LANES)]
                             for cb in range(CB)]
                        sa = tuple(sa[cb] + v[cb] for cb in range(CB))
                        na = tuple(jnp.minimum(na[cb], v[cb])
                                   for cb in range(CB))
                        xa = tuple(jnp.maximum(xa[cb], v[cb])
                                   for cb in range(CB))
                        return sa, na, xa

                    s_acc, n_acc, x_acc = lax.fori_loop(
                        0, k, acc_row,
                        (tuple(s_acc), tuple(n_acc), tuple(x_acc)))

                    @pl.when(jnp.logical_and(run0 >= l_lo, run0 < l_lo + W))
                    def _(s_acc=s_acc, n_acc=n_acc, x_acc=x_acc):
                        flush(run0, s_acc, n_acc, x_acc)

                    vk = tuple(rows_ref[g * 16 + k, pl.ds(cb * LANES, LANES)]
                               for cb in range(CB))
                    s_acc, n_acc, x_acc = lax.fori_loop(
                        k + 1, 16, acc_row, (vk, vk, vk))
                    st_racc(s_acc, n_acc, x_acc)
                    run_s[0] = lab2

                @pl.when(jnp.logical_not(one_b))
                def _():
                    _general_group(rows_ref, li, g)

        def _general_group(rows_ref, li, g):
                run_lab = run_s[0]
                s_acc, n_acc, x_acc = ld_racc()
                for i in range(16):
                    lab = li[i]
                    is_new = lab != run_lab
                    do_flush = jnp.logical_and(
                        is_new,
                        jnp.logical_and(run_lab >= l_lo, run_lab < l_lo + W))

                    @pl.when(do_flush)
                    def _(run_lab=run_lab, s_acc=s_acc, n_acc=n_acc,
                          x_acc=x_acc):
                        flush(run_lab, s_acc, n_acc, x_acc)

                    v = [rows_ref[g * 16 + i, pl.ds(cb * LANES, LANES)]
                         for cb in range(CB)]
                    s_acc = [jnp.where(is_new, v[cb], s_acc[cb] + v[cb])
                             for cb in range(CB)]
                    n_acc = [
                        jnp.minimum(jnp.where(is_new, _INF, n_acc[cb]), v[cb])
                        for cb in range(CB)]
                    x_acc = [
                        jnp.maximum(jnp.where(is_new, -_INF, x_acc[cb]), v[cb])
                        for cb in range(CB)]
                    run_lab = jnp.where(is_new, lab, run_lab)
                st_racc(s_acc, n_acc, x_acc)
                run_s[0] = run_lab

        def chunk_start(i):
            return jnp.minimum(base + i * CHUNK, N - CHUNK)

        H = CHUNK // 2

        def fire(i, b):
            s = chunk_start(i)
            pltpu.async_copy(x_hbm.at[pl.ds(s, H)],
                             rows_refs[b].at[pl.ds(0, H)], sems[b])
            pltpu.async_copy(x_hbm.at[pl.ds(s + H, H)],
                             rows_refs[b].at[pl.ds(H, H)], sems[b])
            pltpu.async_copy(lab_hbm.at[pl.ds(s, CHUNK)], labs_refs[b],
                             sems[b])

        def wait_slot(b):
            pltpu.make_async_copy(
                x_hbm.at[pl.ds(0, CHUNK)], rows_refs[b], sems[b]).wait()
            pltpu.make_async_copy(
                lab_hbm.at[pl.ds(0, CHUNK)], labs_refs[b], sems[b]).wait()

        for b in range(NB):
            @pl.when(b < nch)
            def _(b=b):
                fire(b, b)

        def super_body(ss, c):
            for b in range(NB):
                i = ss * NB + b

                @pl.when(i < nch)
                def _(i=i, b=b):
                    wait_slot(b)
                    g0 = (base + i * CHUNK - chunk_start(i)) // 16

                    def gbody(g, cc, b=b):
                        process_group(rows_refs[b], labs_refs[b], g)
                        return cc

                    lax.fori_loop(g0, GPC, gbody, 0)

                    @pl.when(i + NB < nch)
                    def _(i=i, b=b):
                        fire(i + NB, b)
            return c

        lax.fori_loop(0, (nch + NB - 1) // NB, super_body, 0)

        run_lab = run_s[0]

        @pl.when(jnp.logical_and(run_lab >= l_lo, run_lab < l_lo + W))
        def _():
            s_acc, n_acc, x_acc = ld_racc()
            flush(run_lab, s_acc, n_acc, x_acc)

        pltpu.sync_copy(cnt_hbm.at[pl.ds(l_lo, W)], cnt_v)

        def div_body(g, c):
            cnt = cnt_v[pl.ds(g * 16, 16)]
            rcp = 1.0 / jnp.maximum(cnt.astype(jnp.float32), 1.0)
            for i in range(16):
                r = rcp[i]
                off = (g * 16 + i) * (3 * D)
                for cb in range(CB):
                    sl = pl.ds(off + cb * LANES, LANES)
                    acc_v[sl] = acc_v[sl] * r
            return c

        lax.fori_loop(0, W // 16, div_body, 0)

        @pl.when(job < FULL_JOBS)
        def _():
            pltpu.sync_copy(acc_v, out_hbm.at[pl.ds(job * ACC_W, ACC_W)])

        @pl.when(job == FULL_JOBS)
        def _():
            pltpu.sync_copy(acc_v.at[pl.ds(0, REM_WORDS)],
                            out_hbm.at[pl.ds(FULL_JOBS * ACC_W, REM_WORDS)])
        return 0

    lax.fori_loop(0, JOBS_PER_W, job_body, 0)


def kernel(input, labels, labelcount):
    counts_pad = jnp.concatenate(
        [labelcount, jnp.ones((L_PAD - L,), jnp.int32)])
    starts = jnp.concatenate(
        [jnp.zeros((1,), jnp.int32), jnp.cumsum(labelcount, dtype=jnp.int32)])
    bnd = jnp.minimum(jnp.arange(0, L_PAD + W, W, dtype=jnp.int32), L)
    js = starts[bnd]
    jinfo = (jnp.zeros((JOBS, 16), jnp.int32)
             .at[:, 0].set(js[:-1])
             .at[:, 1].set(js[1:]))

    mesh = plsc.VectorSubcoreMesh(core_axis_name="c", subcore_axis_name="s")
    out_flat = pl.kernel(
        _sc_body,
        out_type=jax.ShapeDtypeStruct((OUT_WORDS,), jnp.float32),
        mesh=mesh,
        scratch_types=[
            pltpu.VMEM((CHUNK, D), jnp.float32),
            pltpu.VMEM((CHUNK, D), jnp.float32),
            pltpu.VMEM((CHUNK,), jnp.int32),
            pltpu.VMEM((CHUNK,), jnp.int32),
            pltpu.VMEM((ACC_W,), jnp.float32),
            pltpu.VMEM((3 * D,), jnp.float32),
            pltpu.VMEM((W,), jnp.int32),
            pltpu.VMEM((16,), jnp.int32),
            pltpu.SMEM((8,), jnp.int32),
            pltpu.SemaphoreType.DMA,
            pltpu.SemaphoreType.DMA,
        ],
    )(input, labels, counts_pad, jinfo)
    return out_flat.reshape(L, 3, D)


# speedup vs baseline: 1.0127x; 1.0053x over previous
import jax
import jax.numpy as jnp
from jax import lax
from jax.experimental import pallas as pl
from jax.experimental.pallas import tpu as pltpu
from jax.experimental.pallas import tpu_sc as plsc

N = 320000
D = 128
L = 10000

NC = 2
NS = 16
LANES = 16
NW = NC * NS
JOBS_PER_W = 2
JOBS = NW * JOBS_PER_W
W = (-(-L // JOBS) + 7) // 8 * 8
L_PAD = JOBS * W
CB = D // LANES
ACC_W = W * 3 * D
OUT_WORDS = L * 3 * D
FULL_JOBS = L // W
REM_WORDS = (L - FULL_JOBS * W) * 3 * D

CHUNK = 128
GPC = CHUNK // LANES
NB = 2

_INF = float("inf")


def _sc_body(x_hbm, lab_hbm, cnt_hbm, jinfo_hbm, out_hbm,
             rows0, rows1, labs0, labs1, acc_v, racc_v, cnt_v, jinfo_v,
             run_s, sem0, sem1, semo):
    wid = lax.axis_index("s") * NC + lax.axis_index("c")
    zeros = jnp.zeros((LANES,), jnp.float32)
    rows_refs = (rows0, rows1)
    labs_refs = (labs0, labs1)
    sems = (sem0, sem1)

    def ld_racc():
        s_acc = [racc_v[pl.ds(cb * LANES, LANES)] for cb in range(CB)]
        n_acc = [racc_v[pl.ds(D + cb * LANES, LANES)] for cb in range(CB)]
        x_acc = [racc_v[pl.ds(2 * D + cb * LANES, LANES)] for cb in range(CB)]
        return s_acc, n_acc, x_acc

    def st_racc(s_acc, n_acc, x_acc):
        for cb in range(CB):
            racc_v[pl.ds(cb * LANES, LANES)] = s_acc[cb]
            racc_v[pl.ds(D + cb * LANES, LANES)] = n_acc[cb]
            racc_v[pl.ds(2 * D + cb * LANES, LANES)] = x_acc[cb]

    def job_body(jj, _):
        job = wid * JOBS_PER_W + jj
        l_lo = job * W

        pltpu.sync_copy(jinfo_hbm.at[job], jinfo_v)
        jv = jinfo_v[...]
        r0 = jv[0]
        r1 = jv[1]
        base = (r0 // 16) * 16
        end = ((r1 + 15) // 16) * 16
        nch = (end - base + CHUNK - 1) // CHUNK

        run_s[0] = jnp.int32(-1)

        def flush(run_lab, s_acc, n_acc, x_acc):
            off = (run_lab - l_lo) * (3 * D)
            for cb in range(CB):
                acc_v[pl.ds(off + cb * LANES, LANES)] = s_acc[cb]
                acc_v[pl.ds(off + D + cb * LANES, LANES)] = n_acc[cb]
                acc_v[pl.ds(off + 2 * D + cb * LANES, LANES)] = x_acc[cb]

        def process_group(rows_ref, labs_ref, g):
            run0 = run_s[0]
            lv = labs_ref[pl.ds(g * 16, 16)]
            all_same = jnp.logical_and(lv[0] == run0, lv[15] == run0)

            @pl.when(all_same)
            def _():
                s_acc, n_acc, x_acc = ld_racc()
                for i in range(16):
                    v = [rows_ref[g * 16 + i, pl.ds(cb * LANES, LANES)]
                         for cb in range(CB)]
                    s_acc = [s_acc[cb] + v[cb] for cb in range(CB)]
                    n_acc = [jnp.minimum(n_acc[cb], v[cb])
                             for cb in range(CB)]
                    x_acc = [jnp.maximum(x_acc[cb], v[cb])
                             for cb in range(CB)]
                st_racc(s_acc, n_acc, x_acc)

            @pl.when(jnp.logical_not(all_same))
            def _():
                lab2 = lv[15]
                li = [lv[i] for i in range(16)]
                eq2 = [jnp.logical_or(li[i] == run0, li[i] == lab2)
                       for i in range(16)]
                while len(eq2) > 1:
                    eq2 = [jnp.logical_and(eq2[j], eq2[j + 1])
                           for j in range(0, len(eq2) - 1, 2)] + (
                               [eq2[-1]] if len(eq2) % 2 else [])
                one_b = eq2[0]
                kv = [jnp.where(li[i] == lab2, jnp.int32(i), jnp.int32(16))
                      for i in range(16)]
                while len(kv) > 1:
                    kv = [jnp.minimum(kv[j], kv[j + 1])
                          for j in range(0, len(kv) - 1, 2)] + (
                              [kv[-1]] if len(kv) % 2 else [])
                k = kv[0]

                @pl.when(one_b)
                def _():
                    s_acc, n_acc, x_acc = ld_racc()

                    def acc_row(i, c):
                        sa, na, xa = c
                        v = [rows_ref[g * 16 + i, pl.ds(cb * LANES, LANES)]
                             for cb in range(CB)]
                        sa = tuple(sa[cb] + v[cb] for cb in range(CB))
                        na = tuple(jnp.minimum(na[cb], v[cb])
                                   for cb in range(CB))
                        xa = tuple(jnp.maximum(xa[cb], v[cb])
                                   for cb in range(CB))
                        return sa, na, xa

                    s_acc, n_acc, x_acc = lax.fori_loop(
                        0, k, acc_row,
                        (tuple(s_acc), tuple(n_acc), tuple(x_acc)))

                    @pl.when(jnp.logical_and(run0 >= l_lo, run0 < l_lo + W))
                    def _(s_acc=s_acc, n_acc=n_acc, x_acc=x_acc):
                        flush(run0, s_acc, n_acc, x_acc)

                    vk = tuple(rows_ref[g * 16 + k, pl.ds(cb * LANES, LANES)]
                               for cb in range(CB))
                    s_acc, n_acc, x_acc = lax.fori_loop(
                        k + 1, 16, acc_row, (vk, vk, vk))
                    st_racc(s_acc, n_acc, x_acc)
                    run_s[0] = lab2

                @pl.when(jnp.logical_not(one_b))
                def _():
                    _general_group(rows_ref, li, g)

        def _general_group(rows_ref, li, g):
                run_lab = run_s[0]
                s_acc, n_acc, x_acc = ld_racc()
                for i in range(16):
                    lab = li[i]
                    is_new = lab != run_lab
                    do_flush = jnp.logical_and(
                        is_new,
                        jnp.logical_and(run_lab >= l_lo, run_lab < l_lo + W))

                    @pl.when(do_flush)
                    def _(run_lab=run_lab, s_acc=s_acc, n_acc=n_acc,
                          x_acc=x_acc):
                        flush(run_lab, s_acc, n_acc, x_acc)

                    v = [rows_ref[g * 16 + i, pl.ds(cb * LANES, LANES)]
                         for cb in range(CB)]
                    s_acc = [jnp.where(is_new, v[cb], s_acc[cb] + v[cb])
                             for cb in range(CB)]
                    n_acc = [
                        jnp.minimum(jnp.where(is_new, _INF, n_acc[cb]), v[cb])
                        for cb in range(CB)]
                    x_acc = [
                        jnp.maximum(jnp.where(is_new, -_INF, x_acc[cb]), v[cb])
                        for cb in range(CB)]
                    run_lab = jnp.where(is_new, lab, run_lab)
                st_racc(s_acc, n_acc, x_acc)
                run_s[0] = run_lab

        def chunk_start(i):
            return jnp.minimum(base + i * CHUNK, N - CHUNK)

        H = CHUNK // 2

        def fire(i, b):
            s = chunk_start(i)
            pltpu.async_copy(x_hbm.at[pl.ds(s, H)],
                             rows_refs[b].at[pl.ds(0, H)], sems[b])
            pltpu.async_copy(x_hbm.at[pl.ds(s + H, H)],
                             rows_refs[b].at[pl.ds(H, H)], sems[b])
            pltpu.async_copy(lab_hbm.at[pl.ds(s, CHUNK)], labs_refs[b],
                             sems[b])

        def wait_slot(b):
            pltpu.make_async_copy(
                x_hbm.at[pl.ds(0, CHUNK)], rows_refs[b], sems[b]).wait()
            pltpu.make_async_copy(
                lab_hbm.at[pl.ds(0, CHUNK)], labs_refs[b], sems[b]).wait()

        for b in range(NB):
            @pl.when(b < nch)
            def _(b=b):
                fire(b, b)

        @pl.when(jj == 1)
        def _():
            pj = job - 1

            @pl.when(pj < FULL_JOBS)
            def _(pj=pj):
                pltpu.make_async_copy(
                    acc_v, out_hbm.at[pl.ds(pj * ACC_W, ACC_W)], semo).wait()

            @pl.when(pj == FULL_JOBS)
            def _(pj=pj):
                pltpu.make_async_copy(
                    acc_v.at[pl.ds(0, REM_WORDS)],
                    out_hbm.at[pl.ds(FULL_JOBS * ACC_W, REM_WORDS)],
                    semo).wait()

        def super_body(ss, c):
            for b in range(NB):
                i = ss * NB + b

                @pl.when(i < nch)
                def _(i=i, b=b):
                    wait_slot(b)
                    g0 = (base + i * CHUNK - chunk_start(i)) // 16

                    def gbody(g, cc, b=b):
                        process_group(rows_refs[b], labs_refs[b], g)
                        return cc

                    lax.fori_loop(g0, GPC, gbody, 0)

                    @pl.when(i + NB < nch)
                    def _(i=i, b=b):
                        fire(i + NB, b)
            return c

        lax.fori_loop(0, (nch + NB - 1) // NB, super_body, 0)

        run_lab = run_s[0]

        @pl.when(jnp.logical_and(run_lab >= l_lo, run_lab < l_lo + W))
        def _():
            s_acc, n_acc, x_acc = ld_racc()
            flush(run_lab, s_acc, n_acc, x_acc)

        pltpu.sync_copy(cnt_hbm.at[pl.ds(l_lo, W)], cnt_v)

        def div_body(g, c):
            cnt = cnt_v[pl.ds(g * 16, 16)]
            rcp = 1.0 / jnp.maximum(cnt.astype(jnp.float32), 1.0)
            for i in range(16):
                r = rcp[i]
                off = (g * 16 + i) * (3 * D)
                present = cnt[i] > 0

                @pl.when(present)
                def _(r=r, off=off):
                    for cb in range(CB):
                        sl = pl.ds(off + cb * LANES, LANES)
                        acc_v[sl] = acc_v[sl] * r

                @pl.when(jnp.logical_not(present))
                def _(off=off):
                    for u in range(3 * CB):
                        acc_v[pl.ds(off + u * LANES, LANES)] = zeros
            return c

        lax.fori_loop(0, W // 16, div_body, 0)

        @pl.when(jj == 0)
        def _():
            @pl.when(job < FULL_JOBS)
            def _():
                pltpu.async_copy(acc_v, out_hbm.at[pl.ds(job * ACC_W, ACC_W)],
                                 semo)

            @pl.when(job == FULL_JOBS)
            def _():
                pltpu.async_copy(acc_v.at[pl.ds(0, REM_WORDS)],
                                 out_hbm.at[pl.ds(FULL_JOBS * ACC_W,
                                                  REM_WORDS)], semo)

        @pl.when(jj == 1)
        def _():
            @pl.when(job < FULL_JOBS)
            def _():
                pltpu.sync_copy(acc_v, out_hbm.at[pl.ds(job * ACC_W, ACC_W)])

            @pl.when(job == FULL_JOBS)
            def _():
                pltpu.sync_copy(acc_v.at[pl.ds(0, REM_WORDS)],
                                out_hbm.at[pl.ds(FULL_JOBS * ACC_W,
                                                 REM_WORDS)])
        return 0

    lax.fori_loop(0, JOBS_PER_W, job_body, 0)


def kernel(input, labels, labelcount):
    counts_pad = jnp.concatenate(
        [labelcount, jnp.ones((L_PAD - L,), jnp.int32)])
    starts = jnp.concatenate(
        [jnp.zeros((1,), jnp.int32), jnp.cumsum(labelcount, dtype=jnp.int32)])
    bnd = jnp.minimum(jnp.arange(0, L_PAD + W, W, dtype=jnp.int32), L)
    js = starts[bnd]
    jinfo = (jnp.zeros((JOBS, 16), jnp.int32)
             .at[:, 0].set(js[:-1])
             .at[:, 1].set(js[1:]))

    mesh = plsc.VectorSubcoreMesh(core_axis_name="c", subcore_axis_name="s")
    out_flat = pl.kernel(
        _sc_body,
        out_type=jax.ShapeDtypeStruct((OUT_WORDS,), jnp.float32),
        mesh=mesh,
        scratch_types=[
            pltpu.VMEM((CHUNK, D), jnp.float32),
            pltpu.VMEM((CHUNK, D), jnp.float32),
            pltpu.VMEM((CHUNK,), jnp.int32),
            pltpu.VMEM((CHUNK,), jnp.int32),
            pltpu.VMEM((ACC_W,), jnp.float32),
            pltpu.VMEM((3 * D,), jnp.float32),
            pltpu.VMEM((W,), jnp.int32),
            pltpu.VMEM((16,), jnp.int32),
            pltpu.SMEM((8,), jnp.int32),
            pltpu.SemaphoreType.DMA,
            pltpu.SemaphoreType.DMA,
            pltpu.SemaphoreType.DMA,
        ],
    )(input, labels, counts_pad, jinfo)
    return out_flat.reshape(L, 3, D)
